# baseline (speedup 1.0000x reference)
import jax
import jax.numpy as jnp
from jax.experimental import pallas as pl
from jax.experimental.pallas import tpu as pltpu

_VERTICES_NUM = (120000, 150000, 100000, 130000, 140000, 110000, 125000, 135000)
_NSEL = 4
_SEL = _VERTICES_NUM[:_NSEL]
_TOTAL = sum(_SEL)
_FDIM = 32
_W = 7680
_WIN = _W + 128
_NBLK = -(-_TOTAL // _W)

_D = []
_d = 0
for _vn in _SEL:
    _D.append(_d)
    _d += _vn
_A = [-(-_D[t] // 128) * 128 for t in range(_NSEL)]
_SH = [_A[t] - _D[t] for t in range(_NSEL)]
_SPECIAL = [_D[t] // _W for t in range(1, _NSEL)] + [_NBLK - 1]
_LO = [0] + [_D[t] // _W + 1 for t in range(1, _NSEL)]
_HI = [_D[t + 1] // _W - 1 for t in range(_NSEL - 1)] + [_NBLK - 2]
for _t in range(_NSEL):
    assert _LO[_t] * _W - _A[_t] >= 0
    assert _HI[_t] * _W - _A[_t] + _WIN <= _SEL[_t]


def _issue(tables, ibuf, sems, j, b):
    for t in range(_NSEL):
        @pl.when(jnp.logical_and(j >= _LO[t], j <= _HI[t]))
        def _(t=t):
            abase = pl.multiple_of(j * _W - _A[t], 128)
            pltpu.make_async_copy(
                tables[t].at[:, pl.ds(abase, _WIN)], ibuf.at[b], sems.at[b]
            ).start()


def _wait(tables, ibuf, sems, b):
    pltpu.make_async_copy(
        tables[0].at[:, pl.ds(0, _WIN)], ibuf.at[b], sems.at[b]).wait()


def _body_a(p0, p1, p2, p3, out_ref, ibuf, sems):
    tables = (p0, p1, p2, p3)
    i = pl.program_id(0)

    def is_spec(j):
        c = j == _SPECIAL[0]
        for s in _SPECIAL[1:]:
            c = jnp.logical_or(c, j == s)
        return c

    @pl.when(i == 0)
    def _():
        _issue(tables, ibuf, sems, jnp.int32(0), 0)

    nxt = i + 1
    for b in (0, 1):
        @pl.when(jnp.logical_and(
            jnp.logical_and(nxt < _NBLK, jnp.logical_not(is_spec(nxt))),
            nxt % 2 == b))
        def _(b=b):
            _issue(tables, ibuf, sems, nxt, b)

    for b in (0, 1):
        @pl.when(jnp.logical_and(
            jnp.logical_and(jnp.logical_not(is_spec(i)), i % 2 == b),
            i < _NBLK))
        def _(b=b):
            _wait(tables, ibuf, sems, b)
            for t in range(_NSEL):
                @pl.when(jnp.logical_and(i >= _LO[t], i <= _HI[t]))
                def _(t=t, b=b):
                    out_ref[...] = ibuf[b, :, _SH[t]: _SH[t] + _W]


def _call_a(p0, p1, p2, p3):
    return pl.pallas_call(
        _body_a,
        grid=(_NBLK,),
        out_shape=jax.ShapeDtypeStruct((_FDIM, _TOTAL), jnp.float32),
        in_specs=[pl.BlockSpec(memory_space=pltpu.MemorySpace.HBM)] * _NSEL,
        out_specs=pl.BlockSpec((_FDIM, _W), lambda i: (0, i)),
        scratch_shapes=[
            pltpu.VMEM((2, _FDIM, _WIN), jnp.float32),
            pltpu.SemaphoreType.DMA((2,)),
        ],
    )(p0, p1, p2, p3)


_PA_W = []
_PB_W = []
for _k, _j in enumerate(_SPECIAL):
    _t = _k
    _PA_W.append(_SEL[_t] - (_j * _W - _D[_t]))
    _PB_W.append(min(_j * _W + _W, _TOTAL) - _D[_t + 1] if _t + 1 < _NSEL else 0)


def _body_b(*refs):
    (idx_ref, vnt_ref, prev, pa0, pb0, pa1, pb1, pa2, pb2, pa3,
     out_ref, vnum_ref) = refs
    i = pl.program_id(0)
    pas = (pa0, pa1, pa2, pa3)
    pbs = (pb0, pb1, pb2, None)
    for k in range(4):
        @pl.when(i == k)
        def _(k=k):
            parts = [pas[k][...]]
            if pbs[k] is not None:
                parts.append(pbs[k][...])
            pad = _W - sum(p.shape[1] for p in parts)
            if pad:
                parts.append(jnp.zeros((_FDIM, pad), jnp.float32))
            out_ref[...] = jnp.concatenate(parts, axis=1)

    @pl.when(i == 0)
    def _():
        for k in range(_NSEL):
            vnum_ref[k] = vnt_ref[idx_ref[k]]


def _call_b(prev, pieces, idx, vnt):
    in_specs = [
        pl.BlockSpec(memory_space=pltpu.MemorySpace.SMEM),
        pl.BlockSpec(memory_space=pltpu.MemorySpace.SMEM),
        pl.BlockSpec(memory_space=pltpu.MemorySpace.HBM),
    ] + [pl.BlockSpec((_FDIM, p.shape[1]), lambda i: (0, 0)) for p in pieces]
    return pl.pallas_call(
        _body_b,
        grid=(4,),
        out_shape=(
            jax.ShapeDtypeStruct((_FDIM, _TOTAL), jnp.float32),
            jax.ShapeDtypeStruct((_NSEL,), jnp.int32),
        ),
        in_specs=in_specs,
        out_specs=(
            pl.BlockSpec((_FDIM, _W), lambda i: (0, jnp.where(
                i == 0, _SPECIAL[0], jnp.where(
                    i == 1, _SPECIAL[1], jnp.where(
                        i == 2, _SPECIAL[2], _SPECIAL[3]))))),
            pl.BlockSpec(memory_space=pltpu.MemorySpace.SMEM),
        ),
        input_output_aliases={2: 0},
    )(idx, vnt, prev, *pieces)


@jax.jit
def _concat(p0, p1, p2, p3, idx, vnt):
    tables = (p0, p1, p2, p3)
    out = _call_a(p0, p1, p2, p3)
    pieces = []
    for k, j in enumerate(_SPECIAL):
        pieces.append(tables[k][:, _SEL[k] - _PA_W[k]:])
        if k + 1 < _NSEL:
            pieces.append(tables[k + 1][:, : _PB_W[k]])
    out, v_num = _call_b(out, pieces, idx, vnt)
    return out, v_num


def kernel(p0, p1, p2, p3, p4, p5, p6, p7, default_features, indexes):
    vnt = jnp.asarray(_VERTICES_NUM, dtype=jnp.int32)
    p_params, v_num = _concat(p0, p1, p2, p3, indexes, vnt)
    return p_params, default_features, v_num


# speedup vs baseline: 1.4040x; 1.4040x over previous
import jax
import jax.numpy as jnp
from jax.experimental import pallas as pl
from jax.experimental.pallas import tpu as pltpu

_VERTICES_NUM = (120000, 150000, 100000, 130000, 140000, 110000, 125000, 135000)
_NSEL = 4
_SEL = _VERTICES_NUM[:_NSEL]
_TOTAL = sum(_SEL)
_FDIM = 32
_W = 7680
_WIN = _W + 128
_NBLK = -(-_TOTAL // _W)
_NB = 4

_D = []
_d = 0
for _vn in _SEL:
    _D.append(_d)
    _d += _vn
_A = [-(-_D[t] // 128) * 128 for t in range(_NSEL)]
_SH = [_A[t] - _D[t] for t in range(_NSEL)]
_SPECIAL = [_D[t] // _W for t in range(1, _NSEL)] + [_NBLK - 1]
_LO = [0] + [_D[t] // _W + 1 for t in range(1, _NSEL)]
_HI = [_D[t + 1] // _W - 1 for t in range(_NSEL - 1)] + [_NBLK - 2]
for _t in range(_NSEL):
    assert _LO[_t] * _W - _A[_t] >= 0
    assert _HI[_t] * _W - _A[_t] + _WIN <= _SEL[_t]


def _issue(tables, ibuf, sems, j, b):
    for t in range(_NSEL):
        @pl.when(jnp.logical_and(j >= _LO[t], j <= _HI[t]))
        def _(t=t):
            abase = pl.multiple_of(j * _W - _A[t], 128)
            pltpu.make_async_copy(
                tables[t].at[:, pl.ds(abase, _WIN)], ibuf.at[b], sems.at[b]
            ).start()


def _wait(tables, ibuf, sems, b):
    pltpu.make_async_copy(
        tables[0].at[:, pl.ds(0, _WIN)], ibuf.at[b], sems.at[b]).wait()


def _body_a(p0, p1, p2, p3, out_ref, ibuf, sems):
    tables = (p0, p1, p2, p3)
    i = pl.program_id(0)

    def is_spec(j):
        c = j == _SPECIAL[0]
        for s in _SPECIAL[1:]:
            c = jnp.logical_or(c, j == s)
        return c

    @pl.when(i == 0)
    def _():
        for j in range(_NB - 1):
            _issue(tables, ibuf, sems, jnp.int32(j), j % _NB)

    nxt = i + (_NB - 1)
    for b in range(_NB):
        @pl.when(jnp.logical_and(
            jnp.logical_and(nxt < _NBLK, jnp.logical_not(is_spec(nxt))),
            nxt % _NB == b))
        def _(b=b):
            _issue(tables, ibuf, sems, nxt, b)

    for b in range(_NB):
        @pl.when(jnp.logical_and(
            jnp.logical_not(is_spec(i)), i % _NB == b))
        def _(b=b):
            _wait(tables, ibuf, sems, b)
            for t in range(_NSEL):
                @pl.when(jnp.logical_and(i >= _LO[t], i <= _HI[t]))
                def _(t=t, b=b):
                    out_ref[...] = ibuf[b, :, _SH[t]: _SH[t] + _W]


def _call_a(p0, p1, p2, p3):
    return pl.pallas_call(
        _body_a,
        grid=(_NBLK,),
        out_shape=jax.ShapeDtypeStruct((_FDIM, _TOTAL), jnp.float32),
        in_specs=[pl.BlockSpec(memory_space=pltpu.MemorySpace.HBM)] * _NSEL,
        out_specs=pl.BlockSpec((_FDIM, _W), lambda i: (0, i)),
        scratch_shapes=[
            pltpu.VMEM((_NB, _FDIM, _WIN), jnp.float32),
            pltpu.SemaphoreType.DMA((_NB,)),
        ],
    )(p0, p1, p2, p3)


_PA_W = []
_PB_W = []
for _k, _j in enumerate(_SPECIAL):
    _t = _k
    _PA_W.append(_SEL[_t] - (_j * _W - _D[_t]))
    _PB_W.append(min(_j * _W + _W, _TOTAL) - _D[_t + 1] if _t + 1 < _NSEL else 0)


def _body_b(*refs):
    (idx_ref, vnt_ref, prev, pa0, pb0, pa1, pb1, pa2, pb2, pa3,
     out_ref, vnum_ref) = refs
    i = pl.program_id(0)
    pas = (pa0, pa1, pa2, pa3)
    pbs = (pb0, pb1, pb2, None)
    for k in range(4):
        @pl.when(i == k)
        def _(k=k):
            parts = [pas[k][...]]
            if pbs[k] is not None:
                parts.append(pbs[k][...])
            pad = _W - sum(p.shape[1] for p in parts)
            if pad:
                parts.append(jnp.zeros((_FDIM, pad), jnp.float32))
            out_ref[...] = jnp.concatenate(parts, axis=1)

    @pl.when(i == 0)
    def _():
        for k in range(_NSEL):
            vnum_ref[k] = vnt_ref[idx_ref[k]]


def _call_b(prev, pieces, idx, vnt):
    in_specs = [
        pl.BlockSpec(memory_space=pltpu.MemorySpace.SMEM),
        pl.BlockSpec(memory_space=pltpu.MemorySpace.SMEM),
        pl.BlockSpec(memory_space=pltpu.MemorySpace.HBM),
    ] + [pl.BlockSpec((_FDIM, p.shape[1]), lambda i: (0, 0)) for p in pieces]
    return pl.pallas_call(
        _body_b,
        grid=(4,),
        out_shape=(
            jax.ShapeDtypeStruct((_FDIM, _TOTAL), jnp.float32),
            jax.ShapeDtypeStruct((_NSEL,), jnp.int32),
        ),
        in_specs=in_specs,
        out_specs=(
            pl.BlockSpec((_FDIM, _W), lambda i: (0, jnp.where(
                i == 0, _SPECIAL[0], jnp.where(
                    i == 1, _SPECIAL[1], jnp.where(
                        i == 2, _SPECIAL[2], _SPECIAL[3]))))),
            pl.BlockSpec(memory_space=pltpu.MemorySpace.SMEM),
        ),
        input_output_aliases={2: 0},
    )(idx, vnt, prev, *pieces)


@jax.jit
def _concat(p0, p1, p2, p3, idx, vnt):
    tables = (p0, p1, p2, p3)
    out = _call_a(p0, p1, p2, p3)
    pieces = []
    for k, j in enumerate(_SPECIAL):
        pieces.append(tables[k][:, _SEL[k] - _PA_W[k]:])
        if k + 1 < _NSEL:
            pieces.append(tables[k + 1][:, : _PB_W[k]])
    out, v_num = _call_b(out, pieces, idx, vnt)
    return out, v_num


def kernel(p0, p1, p2, p3, p4, p5, p6, p7, default_features, indexes):
    vnt = jnp.asarray(_VERTICES_NUM, dtype=jnp.int32)
    p_params, v_num = _concat(p0, p1, p2, p3, indexes, vnt)
    return p_params, default_features, v_num


# speedup vs baseline: 1.4150x; 1.0078x over previous
import jax
import jax.numpy as jnp
from jax.experimental import pallas as pl
from jax.experimental.pallas import tpu as pltpu

_VERTICES_NUM = (120000, 150000, 100000, 130000, 140000, 110000, 125000, 135000)
_NSEL = 4
_SEL = _VERTICES_NUM[:_NSEL]
_TOTAL = sum(_SEL)
_FDIM = 32
_W = 15360
_WIN = _W + 128
_NBLK = -(-_TOTAL // _W)
_NB = 4

_D = []
_d = 0
for _vn in _SEL:
    _D.append(_d)
    _d += _vn
_A = [-(-_D[t] // 128) * 128 for t in range(_NSEL)]
_SH = [_A[t] - _D[t] for t in range(_NSEL)]
_SPECIAL = [_D[t] // _W for t in range(1, _NSEL)] + [_NBLK - 1]
_LO = [0] + [_D[t] // _W + 1 for t in range(1, _NSEL)]
_HI = [_D[t + 1] // _W - 1 for t in range(_NSEL - 1)] + [_NBLK - 2]
for _t in range(_NSEL):
    assert _LO[_t] * _W - _A[_t] >= 0
    assert _HI[_t] * _W - _A[_t] + _WIN <= _SEL[_t]


def _issue(tables, ibuf, sems, j, b):
    for t in range(_NSEL):
        @pl.when(jnp.logical_and(j >= _LO[t], j <= _HI[t]))
        def _(t=t):
            abase = pl.multiple_of(j * _W - _A[t], 128)
            pltpu.make_async_copy(
                tables[t].at[:, pl.ds(abase, _WIN)], ibuf.at[b], sems.at[b]
            ).start()


def _wait(tables, ibuf, sems, b):
    pltpu.make_async_copy(
        tables[0].at[:, pl.ds(0, _WIN)], ibuf.at[b], sems.at[b]).wait()


def _body_a(p0, p1, p2, p3, out_ref, ibuf, sems):
    tables = (p0, p1, p2, p3)
    i = pl.program_id(0)

    def is_spec(j):
        c = j == _SPECIAL[0]
        for s in _SPECIAL[1:]:
            c = jnp.logical_or(c, j == s)
        return c

    @pl.when(i == 0)
    def _():
        for j in range(_NB - 1):
            _issue(tables, ibuf, sems, jnp.int32(j), j % _NB)

    nxt = i + (_NB - 1)
    for b in range(_NB):
        @pl.when(jnp.logical_and(
            jnp.logical_and(nxt < _NBLK, jnp.logical_not(is_spec(nxt))),
            nxt % _NB == b))
        def _(b=b):
            _issue(tables, ibuf, sems, nxt, b)

    for b in range(_NB):
        @pl.when(jnp.logical_and(
            jnp.logical_not(is_spec(i)), i % _NB == b))
        def _(b=b):
            _wait(tables, ibuf, sems, b)
            for t in range(_NSEL):
                @pl.when(jnp.logical_and(i >= _LO[t], i <= _HI[t]))
                def _(t=t, b=b):
                    out_ref[...] = ibuf[b, :, _SH[t]: _SH[t] + _W]


def _call_a(p0, p1, p2, p3):
    return pl.pallas_call(
        _body_a,
        grid=(_NBLK,),
        out_shape=jax.ShapeDtypeStruct((_FDIM, _TOTAL), jnp.float32),
        in_specs=[pl.BlockSpec(memory_space=pltpu.MemorySpace.HBM)] * _NSEL,
        out_specs=pl.BlockSpec((_FDIM, _W), lambda i: (0, i)),
        scratch_shapes=[
            pltpu.VMEM((_NB, _FDIM, _WIN), jnp.float32),
            pltpu.SemaphoreType.DMA((_NB,)),
        ],
    )(p0, p1, p2, p3)


_PA_W = []
_PB_W = []
for _k, _j in enumerate(_SPECIAL):
    _t = _k
    _PA_W.append(_SEL[_t] - (_j * _W - _D[_t]))
    _PB_W.append(min(_j * _W + _W, _TOTAL) - _D[_t + 1] if _t + 1 < _NSEL else 0)


def _body_b(*refs):
    (idx_ref, vnt_ref, prev, pa0, pb0, pa1, pb1, pa2, pb2, pa3,
     out_ref, vnum_ref) = refs
    i = pl.program_id(0)
    pas = (pa0, pa1, pa2, pa3)
    pbs = (pb0, pb1, pb2, None)
    for k in range(4):
        @pl.when(i == k)
        def _(k=k):
            parts = [pas[k][...]]
            if pbs[k] is not None:
                parts.append(pbs[k][...])
            pad = _W - sum(p.shape[1] for p in parts)
            if pad:
                parts.append(jnp.zeros((_FDIM, pad), jnp.float32))
            out_ref[...] = jnp.concatenate(parts, axis=1)

    @pl.when(i == 0)
    def _():
        for k in range(_NSEL):
            vnum_ref[k] = vnt_ref[idx_ref[k]]


def _call_b(prev, pieces, idx, vnt):
    in_specs = [
        pl.BlockSpec(memory_space=pltpu.MemorySpace.SMEM),
        pl.BlockSpec(memory_space=pltpu.MemorySpace.SMEM),
        pl.BlockSpec(memory_space=pltpu.MemorySpace.HBM),
    ] + [pl.BlockSpec((_FDIM, p.shape[1]), lambda i: (0, 0)) for p in pieces]
    return pl.pallas_call(
        _body_b,
        grid=(4,),
        out_shape=(
            jax.ShapeDtypeStruct((_FDIM, _TOTAL), jnp.float32),
            jax.ShapeDtypeStruct((_NSEL,), jnp.int32),
        ),
        in_specs=in_specs,
        out_specs=(
            pl.BlockSpec((_FDIM, _W), lambda i: (0, jnp.where(
                i == 0, _SPECIAL[0], jnp.where(
                    i == 1, _SPECIAL[1], jnp.where(
                        i == 2, _SPECIAL[2], _SPECIAL[3]))))),
            pl.BlockSpec(memory_space=pltpu.MemorySpace.SMEM),
        ),
        input_output_aliases={2: 0},
    )(idx, vnt, prev, *pieces)


@jax.jit
def _concat(p0, p1, p2, p3, idx, vnt):
    tables = (p0, p1, p2, p3)
    out = _call_a(p0, p1, p2, p3)
    pieces = []
    for k, j in enumerate(_SPECIAL):
        pieces.append(tables[k][:, _SEL[k] - _PA_W[k]:])
        if k + 1 < _NSEL:
            pieces.append(tables[k + 1][:, : _PB_W[k]])
    out, v_num = _call_b(out, pieces, idx, vnt)
    return out, v_num


def kernel(p0, p1, p2, p3, p4, p5, p6, p7, default_features, indexes):
    vnt = jnp.asarray(_VERTICES_NUM, dtype=jnp.int32)
    p_params, v_num = _concat(p0, p1, p2, p3, indexes, vnt)
    return p_params, default_features, v_num
